# Initial kernel scaffold; baseline (speedup 1.0000x reference)
import jax
import jax.numpy as jnp
from jax.experimental import pallas as pl


def kernel(x, gn_gamma, gn_beta, w1, b1, w2, b2):
    raise NotImplementedError("write your pallas kernel here")


# speedup vs baseline: 9.3975x; 9.3975x over previous
import functools

import jax
import jax.numpy as jnp
import numpy as np
from jax.experimental import pallas as pl

_B, _C, _H, _W = 2, 96, 224, 224
_WS = 3
_NG = 32
_CPG = _C // _NG
_EPS = 1e-5
_R = 16
_T = _H // _R
_NSTAT = 2 + 2 * _NG


def _stats_body(x_ref, hal_ref, mw_ref, df_ref, st_ref):
    xc = x_ref[0]
    h = hal_ref[0, 0]
    xh = jnp.concatenate([h[:, 0:2], xc, h[:, 2:4]], axis=1)

    xw = jnp.dot(xh.reshape(_C * (_R + 4), _W), mw_ref[...],
                 preferred_element_type=jnp.float32)
    xw = xw.reshape(_C, _R + 4, _W)

    s0 = xw[:, 0:_R]
    s1 = xw[:, 1:_R + 1]
    s2 = xw[:, 2:_R + 2]
    s3 = xw[:, 3:_R + 3]
    s4 = xw[:, 4:_R + 4]
    xeven = 0.125 * (s0 + s1) + 0.375 * (s2 + s3)
    xodd = 0.375 * (s1 + s2) + 0.125 * (s3 + s4)
    r0 = pl.program_id(1) * _R
    rg = r0 + jax.lax.broadcasted_iota(jnp.int32, (1, _R, 1), 1)
    xdu = jnp.where(rg % 2 == 0, xeven, xodd)
    xdu = jnp.where(rg == 0, 0.5 * (s2 + s3), xdu)
    xdu = jnp.where(rg == _H - 1, 0.5 * (s1 + s2), xdu)

    df = jnp.sum(jnp.abs(xc - xdu), axis=0)
    df_ref[0] = df

    xg = xc.reshape(_NG, _CPG, _R, _W)
    gs = jnp.sum(xg, axis=(1, 2, 3))
    gq = jnp.sum(xg * xg, axis=(1, 2, 3))
    st = jnp.concatenate(
        [jnp.min(df).reshape(1), jnp.max(df).reshape(1), gs, gq])
    st_ref[0, 0] = st.reshape(1, _NSTAT)


def _main_body(x_ref, hal_ref, df_ref, st_ref, gam_ref, bet_ref,
               w1_ref, b1_ref, w2_ref, b2_ref, y_ref):
    xc = x_ref[0]
    h = hal_ref[0, 0]
    xh = jnp.concatenate([h[:, 0:1], xc, h[:, 1:2]], axis=1)

    st = st_ref[0]
    dmin = jnp.min(st[:, 0])
    dmax = jnp.max(st[:, 1])
    gsum = jnp.sum(st[:, 2:2 + _NG], axis=0)
    gsq = jnp.sum(st[:, 2 + _NG:], axis=0)
    n_el = float(_CPG * _H * _W)
    mu = gsum / n_el
    var = gsq / n_el - mu * mu

    norm = jnp.sqrt(jnp.sum(xh * xh, axis=0))
    xn = xh / jnp.maximum(norm, 1e-12)[None]
    xhp = jnp.pad(xh, ((0, 0), (0, 0), (1, 1)))
    xnp = jnp.pad(xn, ((0, 0), (0, 0), (1, 1)))
    xcn = xn[:, 1:1 + _R, :]
    sims = []
    for dy in (-1, 0, 1):
        for dx in (-1, 0, 1):
            nb = xnp[:, 1 + dy:1 + dy + _R, 1 + dx:1 + dx + _W]
            sims.append(jnp.sum(xcn * nb, axis=0))
    sims = jnp.stack(sims)

    df = df_ref[0]
    dn = (df - dmin) / (dmax - dmin + 1e-8)
    dp = dn ** 4
    thr = 0.9
    mask = (dp > thr).astype(jnp.float32)
    above = jnp.round((dp - thr) / (1.0 - thr + 1e-8) * 15.0)
    counts = 1.0 + jnp.maximum(above, 0.0) * mask
    k = jnp.minimum(counts.astype(jnp.int32), _WS * _WS)

    gt = (sims[:, None] > sims[None, :]).astype(jnp.int32)
    lidx = jax.lax.broadcasted_iota(jnp.int32, (9, 9, 1, 1), 0)
    jidx = jax.lax.broadcasted_iota(jnp.int32, (9, 9, 1, 1), 1)
    eq = jnp.logical_and(sims[:, None] == sims[None, :],
                         lidx < jidx).astype(jnp.int32)
    rank = jnp.sum(gt + eq, axis=0)
    sel = (rank < k[None]).astype(jnp.float32)
    wts = jnp.exp(sims) * sel
    wts = wts / (jnp.sum(wts, axis=0) + 1e-8)

    out = jnp.zeros((_C, _R, _W), jnp.float32)
    j = 0
    for dy in (-1, 0, 1):
        for dx in (-1, 0, 1):
            nb = xhp[:, 1 + dy:1 + dy + _R, 1 + dx:1 + dx + _W]
            out = out + wts[j][None] * nb
            j += 1

    gam = gam_ref[...]
    bet = bet_ref[...]
    inv = 1.0 / jnp.sqrt(var + _EPS)
    scale = gam * inv[:, None]
    shift = bet - mu[:, None] * scale
    xg = xc.reshape(_NG, _CPG, _R, _W)
    gn4 = xg * scale[:, :, None, None] + shift[:, :, None, None]
    e = out + gn4.reshape(_C, _R, _W)

    ef = e.reshape(_C, _R * _W)
    hid = jnp.dot(w1_ref[...], ef, preferred_element_type=jnp.float32)
    hid = jnp.maximum(hid + b1_ref[0][:, None], 0.0)
    ffn = jnp.dot(w2_ref[...], hid, preferred_element_type=jnp.float32)
    ffn = ffn + b2_ref[0][:, None]
    y_ref[0] = (ef + ffn).reshape(_C, _R, _W)


def _col_resample_matrix():
    half = _W // 2
    d = np.zeros((half, _W), np.float32)
    for n in range(half):
        d[n, 2 * n] = 0.5
        d[n, 2 * n + 1] = 0.5
    u = np.zeros((_W, half), np.float32)
    for i in range(_W):
        src = (i + 0.5) * 0.5 - 0.5
        src = min(max(src, 0.0), half - 1.0)
        i0 = int(np.floor(src))
        i1 = min(i0 + 1, half - 1)
        w = src - i0
        u[i, i0] += 1.0 - w
        u[i, i1] += w
    return jnp.asarray((u @ d).T)


@jax.jit
def kernel(x, gn_gamma, gn_beta, w1, b1, w2, b2):
    mw = _col_resample_matrix()
    xp2 = jnp.pad(x, ((0, 0), (0, 0), (2, 2), (0, 0)))
    hal2 = jnp.stack(
        [jnp.concatenate([xp2[:, :, t * _R:t * _R + 2],
                          xp2[:, :, t * _R + _R + 2:t * _R + _R + 4]], axis=2)
         for t in range(_T)], axis=1)
    xp1 = jnp.pad(x, ((0, 0), (0, 0), (1, 1), (0, 0)))
    hal1 = jnp.stack(
        [jnp.concatenate([xp1[:, :, t * _R:t * _R + 1],
                          xp1[:, :, t * _R + _R + 1:t * _R + _R + 2]], axis=2)
         for t in range(_T)], axis=1)

    df, stats = pl.pallas_call(
        _stats_body,
        grid=(_B, _T),
        in_specs=[
            pl.BlockSpec((1, _C, _R, _W), lambda b, t: (b, 0, t, 0)),
            pl.BlockSpec((1, 1, _C, 4, _W), lambda b, t: (b, t, 0, 0, 0)),
            pl.BlockSpec((_W, _W), lambda b, t: (0, 0)),
        ],
        out_specs=[
            pl.BlockSpec((1, _R, _W), lambda b, t: (b, t, 0)),
            pl.BlockSpec((1, 1, 1, _NSTAT), lambda b, t: (b, t, 0, 0)),
        ],
        out_shape=[
            jax.ShapeDtypeStruct((_B, _H, _W), jnp.float32),
            jax.ShapeDtypeStruct((_B, _T, 1, _NSTAT), jnp.float32),
        ],
    )(x, hal2, mw)
    stats = stats.reshape(_B, _T, _NSTAT)

    y = pl.pallas_call(
        _main_body,
        grid=(_B, _T),
        in_specs=[
            pl.BlockSpec((1, _C, _R, _W), lambda b, t: (b, 0, t, 0)),
            pl.BlockSpec((1, 1, _C, 2, _W), lambda b, t: (b, t, 0, 0, 0)),
            pl.BlockSpec((1, _R, _W), lambda b, t: (b, t, 0)),
            pl.BlockSpec((1, _T, _NSTAT), lambda b, t: (b, 0, 0)),
            pl.BlockSpec((_NG, _CPG), lambda b, t: (0, 0)),
            pl.BlockSpec((_NG, _CPG), lambda b, t: (0, 0)),
            pl.BlockSpec((2 * _C, _C), lambda b, t: (0, 0)),
            pl.BlockSpec((1, 2 * _C), lambda b, t: (0, 0)),
            pl.BlockSpec((_C, 2 * _C), lambda b, t: (0, 0)),
            pl.BlockSpec((1, _C), lambda b, t: (0, 0)),
        ],
        out_specs=pl.BlockSpec((1, _C, _R, _W), lambda b, t: (b, 0, t, 0)),
        out_shape=jax.ShapeDtypeStruct((_B, _C, _H, _W), jnp.float32),
    )(x, hal1, df, stats, gn_gamma.reshape(_NG, _CPG),
      gn_beta.reshape(_NG, _CPG), w1, b1.reshape(1, 2 * _C), w2,
      b2.reshape(1, _C))
    return y


# speedup vs baseline: 11.5905x; 1.2334x over previous
import functools

import jax
import jax.numpy as jnp
import numpy as np
from jax.experimental import pallas as pl

_B, _C, _H, _W = 2, 96, 224, 224
_WS = 3
_NG = 32
_CPG = _C // _NG
_EPS = 1e-5
_R = 16
_T = _H // _R
_NSTAT = 2 + 2 * _NG


def _stats_body(x_ref, hal_ref, mw_ref, df_ref, st_ref):
    xc = x_ref[0]
    h = hal_ref[0, 0]
    xh = jnp.concatenate([h[:, 0:2], xc, h[:, 2:4]], axis=1)

    xw = jnp.dot(xh.reshape(_C * (_R + 4), _W), mw_ref[...],
                 preferred_element_type=jnp.float32)
    xw = xw.reshape(_C, _R + 4, _W)

    r0 = pl.program_id(1) * _R
    rg = r0 + jax.lax.broadcasted_iota(jnp.int32, (1, _R, 1), 1)
    even = rg % 2 == 0
    c0 = jnp.where(even, 0.125, 0.0)
    c1 = jnp.where(even, 0.125, 0.375)
    c2 = jnp.where(even, 0.375, 0.375)
    c3 = jnp.where(even, 0.375, 0.125)
    c4 = jnp.where(even, 0.0, 0.125)
    zero = jnp.zeros_like(c0)
    c0 = jnp.where(rg == 0, zero, c0)
    c1 = jnp.where(rg == 0, zero, jnp.where(rg == _H - 1, 0.5, c1))
    c2 = jnp.where(rg == 0, 0.5, jnp.where(rg == _H - 1, 0.5, c2))
    c3 = jnp.where(rg == 0, 0.5, jnp.where(rg == _H - 1, zero, c3))
    c4 = jnp.where(rg == _H - 1, zero, c4)
    xdu = (c0 * xw[:, 0:_R] + c1 * xw[:, 1:_R + 1] + c2 * xw[:, 2:_R + 2]
           + c3 * xw[:, 3:_R + 3] + c4 * xw[:, 4:_R + 4])

    df = jnp.sum(jnp.abs(xc - xdu), axis=0)
    df_ref[0] = df

    xg = xc.reshape(_NG, _CPG, _R, _W)
    gs = jnp.sum(xg, axis=(1, 2, 3))
    gq = jnp.sum(xg * xg, axis=(1, 2, 3))
    st = jnp.concatenate(
        [jnp.min(df).reshape(1), jnp.max(df).reshape(1), gs, gq])
    st_ref[0, 0] = st.reshape(1, _NSTAT)


def _main_body(x_ref, hal_ref, df_ref, st_ref, gam_ref, bet_ref,
               w1_ref, b1_ref, w2_ref, b2_ref, y_ref):
    xc = x_ref[0]
    h = hal_ref[0, 0]
    xh = jnp.concatenate([h[:, 1:2], xc, h[:, 2:3]], axis=1)

    st = st_ref[0]
    dmin = jnp.min(st[:, 0])
    dmax = jnp.max(st[:, 1])
    gsum = jnp.sum(st[:, 2:2 + _NG], axis=0)
    gsq = jnp.sum(st[:, 2 + _NG:], axis=0)
    n_el = float(_CPG * _H * _W)
    mu = gsum / n_el
    var = gsq / n_el - mu * mu

    inv = 1.0 / jnp.maximum(jnp.sqrt(jnp.sum(xh * xh, axis=0)), 1e-12)
    xhp = jnp.pad(xh, ((0, 0), (0, 0), (1, 1)))
    invp = jnp.pad(inv, ((0, 0), (1, 1)))
    inv_c = inv[1:1 + _R, :]
    sims = []
    for dy in (-1, 0, 1):
        for dx in (-1, 0, 1):
            nb = xhp[:, 1 + dy:1 + dy + _R, 1 + dx:1 + dx + _W]
            dot = jnp.sum(xc * nb, axis=0)
            sims.append(dot * inv_c * invp[1 + dy:1 + dy + _R,
                                           1 + dx:1 + dx + _W])
    sims = jnp.stack(sims)

    df = df_ref[0]
    dn = (df - dmin) / (dmax - dmin + 1e-8)
    dp = dn ** 4
    thr = 0.9
    mask = (dp > thr).astype(jnp.float32)
    above = jnp.round((dp - thr) / (1.0 - thr + 1e-8) * 15.0)
    counts = 1.0 + jnp.maximum(above, 0.0) * mask
    k = jnp.minimum(counts.astype(jnp.int32), _WS * _WS)

    gt = (sims[:, None] > sims[None, :]).astype(jnp.int32)
    lidx = jax.lax.broadcasted_iota(jnp.int32, (9, 9, 1, 1), 0)
    jidx = jax.lax.broadcasted_iota(jnp.int32, (9, 9, 1, 1), 1)
    eq = jnp.logical_and(sims[:, None] == sims[None, :],
                         lidx < jidx).astype(jnp.int32)
    rank = jnp.sum(gt + eq, axis=0)
    sel = (rank < k[None]).astype(jnp.float32)
    wts = jnp.exp(sims) * sel
    wts = wts / (jnp.sum(wts, axis=0) + 1e-8)

    out = jnp.zeros((_C, _R, _W), jnp.float32)
    j = 0
    for dy in (-1, 0, 1):
        for dx in (-1, 0, 1):
            nb = xhp[:, 1 + dy:1 + dy + _R, 1 + dx:1 + dx + _W]
            out = out + wts[j][None] * nb
            j += 1

    gam = gam_ref[...]
    bet = bet_ref[...]
    inv = 1.0 / jnp.sqrt(var + _EPS)
    scale = gam * inv[:, None]
    shift = bet - mu[:, None] * scale
    xg = xc.reshape(_NG, _CPG, _R, _W)
    gn4 = xg * scale[:, :, None, None] + shift[:, :, None, None]
    e = out + gn4.reshape(_C, _R, _W)

    hid = jax.lax.dot_general(w1_ref[...], e, (((1,), (0,)), ((), ())),
                              preferred_element_type=jnp.float32)
    hid = jnp.maximum(hid + b1_ref[0][:, None, None], 0.0)
    ffn = jax.lax.dot_general(w2_ref[...], hid, (((1,), (0,)), ((), ())),
                              preferred_element_type=jnp.float32)
    y_ref[0] = e + ffn + b2_ref[0][:, None, None]


def _col_resample_matrix():
    half = _W // 2
    d = np.zeros((half, _W), np.float32)
    for n in range(half):
        d[n, 2 * n] = 0.5
        d[n, 2 * n + 1] = 0.5
    u = np.zeros((_W, half), np.float32)
    for i in range(_W):
        src = (i + 0.5) * 0.5 - 0.5
        src = min(max(src, 0.0), half - 1.0)
        i0 = int(np.floor(src))
        i1 = min(i0 + 1, half - 1)
        w = src - i0
        u[i, i0] += 1.0 - w
        u[i, i1] += w
    return jnp.asarray((u @ d).T)


@jax.jit
def kernel(x, gn_gamma, gn_beta, w1, b1, w2, b2):
    mw = _col_resample_matrix()
    z2 = jnp.zeros((_B, _C, 2, _W), x.dtype)
    hal2 = jnp.stack(
        [jnp.concatenate(
            [x[:, :, t * _R - 2:t * _R] if t > 0 else z2,
             x[:, :, t * _R + _R:t * _R + _R + 2] if t < _T - 1 else z2],
            axis=2)
         for t in range(_T)], axis=1)

    df, stats = pl.pallas_call(
        _stats_body,
        grid=(_B, _T),
        in_specs=[
            pl.BlockSpec((1, _C, _R, _W), lambda b, t: (b, 0, t, 0)),
            pl.BlockSpec((1, 1, _C, 4, _W), lambda b, t: (b, t, 0, 0, 0)),
            pl.BlockSpec((_W, _W), lambda b, t: (0, 0)),
        ],
        out_specs=[
            pl.BlockSpec((1, _R, _W), lambda b, t: (b, t, 0)),
            pl.BlockSpec((1, 1, 1, _NSTAT), lambda b, t: (b, t, 0, 0)),
        ],
        out_shape=[
            jax.ShapeDtypeStruct((_B, _H, _W), jnp.float32),
            jax.ShapeDtypeStruct((_B, _T, 1, _NSTAT), jnp.float32),
        ],
    )(x, hal2, mw)
    stats = stats.reshape(_B, _T, _NSTAT)

    y = pl.pallas_call(
        _main_body,
        grid=(_B, _T),
        in_specs=[
            pl.BlockSpec((1, _C, _R, _W), lambda b, t: (b, 0, t, 0)),
            pl.BlockSpec((1, 1, _C, 4, _W), lambda b, t: (b, t, 0, 0, 0)),
            pl.BlockSpec((1, _R, _W), lambda b, t: (b, t, 0)),
            pl.BlockSpec((1, _T, _NSTAT), lambda b, t: (b, 0, 0)),
            pl.BlockSpec((_NG, _CPG), lambda b, t: (0, 0)),
            pl.BlockSpec((_NG, _CPG), lambda b, t: (0, 0)),
            pl.BlockSpec((2 * _C, _C), lambda b, t: (0, 0)),
            pl.BlockSpec((1, 2 * _C), lambda b, t: (0, 0)),
            pl.BlockSpec((_C, 2 * _C), lambda b, t: (0, 0)),
            pl.BlockSpec((1, _C), lambda b, t: (0, 0)),
        ],
        out_specs=pl.BlockSpec((1, _C, _R, _W), lambda b, t: (b, 0, t, 0)),
        out_shape=jax.ShapeDtypeStruct((_B, _C, _H, _W), jnp.float32),
    )(x, hal2, df, stats, gn_gamma.reshape(_NG, _CPG),
      gn_beta.reshape(_NG, _CPG), w1, b1.reshape(1, 2 * _C), w2,
      b2.reshape(1, _C))
    return y


# speedup vs baseline: 12.6393x; 1.0905x over previous
import functools

import jax
import jax.numpy as jnp
import numpy as np
from jax.experimental import pallas as pl

_B, _C, _H, _W = 2, 96, 224, 224
_WS = 3
_NG = 32
_CPG = _C // _NG
_EPS = 1e-5
_R = 32
_T = _H // _R
_NSTAT = 2 + 2 * _NG


def _stats_body(x_ref, hal_ref, mw_ref, df_ref, st_ref):
    xc = x_ref[0]
    h = hal_ref[0, 0]
    xh = jnp.concatenate([h[:, 0:2], xc, h[:, 2:4]], axis=1)

    xw = jnp.dot(xh.reshape(_C * (_R + 4), _W), mw_ref[...],
                 preferred_element_type=jnp.float32)
    xw = xw.reshape(_C, _R + 4, _W)

    r0 = pl.program_id(1) * _R
    rg = r0 + jax.lax.broadcasted_iota(jnp.int32, (1, _R, 1), 1)
    even = rg % 2 == 0
    c0 = jnp.where(even, 0.125, 0.0)
    c1 = jnp.where(even, 0.125, 0.375)
    c2 = jnp.where(even, 0.375, 0.375)
    c3 = jnp.where(even, 0.375, 0.125)
    c4 = jnp.where(even, 0.0, 0.125)
    zero = jnp.zeros_like(c0)
    c0 = jnp.where(rg == 0, zero, c0)
    c1 = jnp.where(rg == 0, zero, jnp.where(rg == _H - 1, 0.5, c1))
    c2 = jnp.where(rg == 0, 0.5, jnp.where(rg == _H - 1, 0.5, c2))
    c3 = jnp.where(rg == 0, 0.5, jnp.where(rg == _H - 1, zero, c3))
    c4 = jnp.where(rg == _H - 1, zero, c4)
    xdu = (c0 * xw[:, 0:_R] + c1 * xw[:, 1:_R + 1] + c2 * xw[:, 2:_R + 2]
           + c3 * xw[:, 3:_R + 3] + c4 * xw[:, 4:_R + 4])

    df = jnp.sum(jnp.abs(xc - xdu), axis=0)
    df_ref[0] = df

    xg = xc.reshape(_NG, _CPG, _R, _W)
    gs = jnp.sum(xg, axis=(1, 2, 3))
    gq = jnp.sum(xg * xg, axis=(1, 2, 3))
    st = jnp.concatenate(
        [jnp.min(df).reshape(1), jnp.max(df).reshape(1), gs, gq])
    st_ref[0, 0] = st.reshape(1, _NSTAT)


def _main_body(x_ref, hal_ref, df_ref, st_ref, gam_ref, bet_ref,
               w1_ref, b1_ref, w2_ref, b2_ref, y_ref):
    xc = x_ref[0]
    h = hal_ref[0, 0]
    xh = jnp.concatenate([h[:, 1:2], xc, h[:, 2:3]], axis=1)

    st = st_ref[0]
    dmin = jnp.min(st[:, 0])
    dmax = jnp.max(st[:, 1])
    gsum = jnp.sum(st[:, 2:2 + _NG], axis=0)
    gsq = jnp.sum(st[:, 2 + _NG:], axis=0)
    n_el = float(_CPG * _H * _W)
    mu = gsum / n_el
    var = gsq / n_el - mu * mu

    inv = 1.0 / jnp.maximum(jnp.sqrt(jnp.sum(xh * xh, axis=0)), 1e-12)
    xhp = jnp.pad(xh, ((0, 0), (0, 0), (1, 1)))
    invp = jnp.pad(inv, ((0, 0), (1, 1)))
    inv_c = inv[1:1 + _R, :]
    sims = []
    for dy in (-1, 0, 1):
        for dx in (-1, 0, 1):
            nb = xhp[:, 1 + dy:1 + dy + _R, 1 + dx:1 + dx + _W]
            dot = jnp.sum(xc * nb, axis=0)
            sims.append(dot * inv_c * invp[1 + dy:1 + dy + _R,
                                           1 + dx:1 + dx + _W])
    sims = jnp.stack(sims)

    df = df_ref[0]
    dn = (df - dmin) / (dmax - dmin + 1e-8)
    dp = dn ** 4
    thr = 0.9
    mask = (dp > thr).astype(jnp.float32)
    above = jnp.round((dp - thr) / (1.0 - thr + 1e-8) * 15.0)
    counts = 1.0 + jnp.maximum(above, 0.0) * mask
    k = jnp.minimum(counts.astype(jnp.int32), _WS * _WS)

    gt = (sims[:, None] > sims[None, :]).astype(jnp.int32)
    lidx = jax.lax.broadcasted_iota(jnp.int32, (9, 9, 1, 1), 0)
    jidx = jax.lax.broadcasted_iota(jnp.int32, (9, 9, 1, 1), 1)
    eq = jnp.logical_and(sims[:, None] == sims[None, :],
                         lidx < jidx).astype(jnp.int32)
    rank = jnp.sum(gt + eq, axis=0)
    sel = (rank < k[None]).astype(jnp.float32)
    wts = jnp.exp(sims) * sel
    wts = wts / (jnp.sum(wts, axis=0) + 1e-8)

    out = jnp.zeros((_C, _R, _W), jnp.float32)
    j = 0
    for dy in (-1, 0, 1):
        for dx in (-1, 0, 1):
            nb = xhp[:, 1 + dy:1 + dy + _R, 1 + dx:1 + dx + _W]
            out = out + wts[j][None] * nb
            j += 1

    gam = gam_ref[...]
    bet = bet_ref[...]
    inv = 1.0 / jnp.sqrt(var + _EPS)
    scale = gam * inv[:, None]
    shift = bet - mu[:, None] * scale
    xg = xc.reshape(_NG, _CPG, _R, _W)
    gn4 = xg * scale[:, :, None, None] + shift[:, :, None, None]
    e = out + gn4.reshape(_C, _R, _W)

    hid = jax.lax.dot_general(w1_ref[...], e, (((1,), (0,)), ((), ())),
                              preferred_element_type=jnp.float32)
    hid = jnp.maximum(hid + b1_ref[0][:, None, None], 0.0)
    ffn = jax.lax.dot_general(w2_ref[...], hid, (((1,), (0,)), ((), ())),
                              preferred_element_type=jnp.float32)
    y_ref[0] = e + ffn + b2_ref[0][:, None, None]


def _col_resample_matrix():
    half = _W // 2
    d = np.zeros((half, _W), np.float32)
    for n in range(half):
        d[n, 2 * n] = 0.5
        d[n, 2 * n + 1] = 0.5
    u = np.zeros((_W, half), np.float32)
    for i in range(_W):
        src = (i + 0.5) * 0.5 - 0.5
        src = min(max(src, 0.0), half - 1.0)
        i0 = int(np.floor(src))
        i1 = min(i0 + 1, half - 1)
        w = src - i0
        u[i, i0] += 1.0 - w
        u[i, i1] += w
    return jnp.asarray((u @ d).T)


@jax.jit
def kernel(x, gn_gamma, gn_beta, w1, b1, w2, b2):
    mw = _col_resample_matrix()
    z2 = jnp.zeros((_B, _C, 2, _W), x.dtype)
    hal2 = jnp.stack(
        [jnp.concatenate(
            [x[:, :, t * _R - 2:t * _R] if t > 0 else z2,
             x[:, :, t * _R + _R:t * _R + _R + 2] if t < _T - 1 else z2],
            axis=2)
         for t in range(_T)], axis=1)

    df, stats = pl.pallas_call(
        _stats_body,
        grid=(_B, _T),
        in_specs=[
            pl.BlockSpec((1, _C, _R, _W), lambda b, t: (b, 0, t, 0)),
            pl.BlockSpec((1, 1, _C, 4, _W), lambda b, t: (b, t, 0, 0, 0)),
            pl.BlockSpec((_W, _W), lambda b, t: (0, 0)),
        ],
        out_specs=[
            pl.BlockSpec((1, _R, _W), lambda b, t: (b, t, 0)),
            pl.BlockSpec((1, 1, 1, _NSTAT), lambda b, t: (b, t, 0, 0)),
        ],
        out_shape=[
            jax.ShapeDtypeStruct((_B, _H, _W), jnp.float32),
            jax.ShapeDtypeStruct((_B, _T, 1, _NSTAT), jnp.float32),
        ],
    )(x, hal2, mw)
    stats = stats.reshape(_B, _T, _NSTAT)

    y = pl.pallas_call(
        _main_body,
        grid=(_B, _T),
        in_specs=[
            pl.BlockSpec((1, _C, _R, _W), lambda b, t: (b, 0, t, 0)),
            pl.BlockSpec((1, 1, _C, 4, _W), lambda b, t: (b, t, 0, 0, 0)),
            pl.BlockSpec((1, _R, _W), lambda b, t: (b, t, 0)),
            pl.BlockSpec((1, _T, _NSTAT), lambda b, t: (b, 0, 0)),
            pl.BlockSpec((_NG, _CPG), lambda b, t: (0, 0)),
            pl.BlockSpec((_NG, _CPG), lambda b, t: (0, 0)),
            pl.BlockSpec((2 * _C, _C), lambda b, t: (0, 0)),
            pl.BlockSpec((1, 2 * _C), lambda b, t: (0, 0)),
            pl.BlockSpec((_C, 2 * _C), lambda b, t: (0, 0)),
            pl.BlockSpec((1, _C), lambda b, t: (0, 0)),
        ],
        out_specs=pl.BlockSpec((1, _C, _R, _W), lambda b, t: (b, 0, t, 0)),
        out_shape=jax.ShapeDtypeStruct((_B, _C, _H, _W), jnp.float32),
    )(x, hal2, df, stats, gn_gamma.reshape(_NG, _CPG),
      gn_beta.reshape(_NG, _CPG), w1, b1.reshape(1, 2 * _C), w2,
      b2.reshape(1, _C))
    return y


# speedup vs baseline: 15.8556x; 1.2545x over previous
import functools

import jax
import jax.numpy as jnp
import numpy as np
from jax.experimental import pallas as pl

_B, _C, _H, _W = 2, 96, 224, 224
_WS = 3
_NG = 32
_CPG = _C // _NG
_EPS = 1e-5
_R = 32
_T = _H // _R
_NSTAT = 2 + 2 * _NG


def _stats_body(x_ref, hal_ref, mw_ref, df_ref, st_ref):
    xc = x_ref[0]
    h = hal_ref[0, 0]
    xh = jnp.concatenate([h[:, 0:2], xc, h[:, 2:4]], axis=1)

    xw = jnp.dot(xh.reshape(_C * (_R + 4), _W), mw_ref[...],
                 preferred_element_type=jnp.float32)
    xw = xw.reshape(_C, _R + 4, _W)

    r0 = pl.program_id(1) * _R
    rg = r0 + jax.lax.broadcasted_iota(jnp.int32, (1, _R, 1), 1)
    even = rg % 2 == 0
    c0 = jnp.where(even, 0.125, 0.0)
    c1 = jnp.where(even, 0.125, 0.375)
    c2 = jnp.where(even, 0.375, 0.375)
    c3 = jnp.where(even, 0.375, 0.125)
    c4 = jnp.where(even, 0.0, 0.125)
    zero = jnp.zeros_like(c0)
    c0 = jnp.where(rg == 0, zero, c0)
    c1 = jnp.where(rg == 0, zero, jnp.where(rg == _H - 1, 0.5, c1))
    c2 = jnp.where(rg == 0, 0.5, jnp.where(rg == _H - 1, 0.5, c2))
    c3 = jnp.where(rg == 0, 0.5, jnp.where(rg == _H - 1, zero, c3))
    c4 = jnp.where(rg == _H - 1, zero, c4)
    xdu = (c0 * xw[:, 0:_R] + c1 * xw[:, 1:_R + 1] + c2 * xw[:, 2:_R + 2]
           + c3 * xw[:, 3:_R + 3] + c4 * xw[:, 4:_R + 4])

    df = jnp.sum(jnp.abs(xc - xdu), axis=0)
    df_ref[0] = df

    xg = xc.reshape(_NG, _CPG, _R, _W)
    gs = jnp.sum(xg, axis=(1, 2, 3))
    gq = jnp.sum(xg * xg, axis=(1, 2, 3))
    st = jnp.concatenate(
        [jnp.min(df).reshape(1), jnp.max(df).reshape(1), gs, gq])
    st_ref[0, 0] = st.reshape(1, _NSTAT)


def _main_body(x_ref, hal_ref, df_ref, st_ref, gam_ref, bet_ref,
               w1_ref, b1_ref, w2_ref, b2_ref, y_ref):
    xc = x_ref[0]
    h = hal_ref[0, 0]
    nbm = jnp.concatenate([h[:, 1:2], xc[:, :_R - 1]], axis=1)
    nbp = jnp.concatenate([xc[:, 1:], h[:, 2:3]], axis=1)
    nbs = (nbm, xc, nbp)

    st = st_ref[0]
    dmin = jnp.min(st[:, 0])
    dmax = jnp.max(st[:, 1])
    gsum = jnp.sum(st[:, 2:2 + _NG], axis=0)
    gsq = jnp.sum(st[:, 2 + _NG:], axis=0)
    n_el = float(_CPG * _H * _W)
    mu = gsum / n_el
    var = gsq / n_el - mu * mu

    invs = [1.0 / jnp.maximum(jnp.sqrt(jnp.sum(nb * nb, axis=0)), 1e-12)
            for nb in nbs]
    inv_c = invs[1]
    xcs = [jnp.pad(xc, ((0, 0), (0, 0), (0, 1)))[:, :, 1:],
           xc,
           jnp.pad(xc, ((0, 0), (0, 0), (1, 0)))[:, :, :_W]]
    sims = []
    for iy in range(3):
        nb = nbs[iy]
        ninv_p = jnp.pad(invs[iy], ((0, 0), (1, 1)))
        for ix, dx in enumerate((-1, 0, 1)):
            g = jnp.sum(xcs[ix] * nb, axis=0)
            gp = jnp.pad(g, ((0, 0), (1, 1)))
            dot = gp[:, 1 + dx:1 + dx + _W]
            sims.append(dot * inv_c * ninv_p[:, 1 + dx:1 + dx + _W])
    sims = jnp.stack(sims)

    df = df_ref[0]
    dn = (df - dmin) / (dmax - dmin + 1e-8)
    dp = dn ** 4
    thr = 0.9
    mask = (dp > thr).astype(jnp.float32)
    above = jnp.round((dp - thr) / (1.0 - thr + 1e-8) * 15.0)
    counts = 1.0 + jnp.maximum(above, 0.0) * mask

    kf = counts
    ranks = [jnp.zeros((_R, _W), jnp.float32) for _ in range(9)]
    for a in range(9):
        for b in range(a + 1, 9):
            a_wins = (sims[a] >= sims[b]).astype(jnp.float32)
            ranks[b] = ranks[b] + a_wins
            ranks[a] = ranks[a] + (1.0 - a_wins)
    rank = jnp.stack(ranks)
    sel = (rank < kf[None]).astype(jnp.float32)
    wts = jnp.exp(sims) * sel
    wts = wts / (jnp.sum(wts, axis=0) + 1e-8)

    wp = jnp.pad(wts, ((0, 0), (0, 0), (1, 1)))
    acc = []
    for ix, dx in enumerate((-1, 0, 1)):
        a = jnp.zeros((_C, _R, _W), jnp.float32)
        for iy in range(3):
            w_s = wp[iy * 3 + ix, :, 1 - dx:1 - dx + _W]
            a = a + w_s[None] * nbs[iy]
        if dx == 0:
            acc.append(a)
        else:
            ap = jnp.pad(a, ((0, 0), (0, 0), (1, 1)))
            acc.append(ap[:, :, 1 + dx:1 + dx + _W])
    out = acc[0] + acc[1] + acc[2]

    gam = gam_ref[...]
    bet = bet_ref[...]
    inv = 1.0 / jnp.sqrt(var + _EPS)
    scale = gam * inv[:, None]
    shift = bet - mu[:, None] * scale
    xg = xc.reshape(_NG, _CPG, _R, _W)
    gn4 = xg * scale[:, :, None, None] + shift[:, :, None, None]
    e = out + gn4.reshape(_C, _R, _W)

    ef = e.reshape(_C, _R * _W)
    hid = jnp.dot(w1_ref[...], ef, preferred_element_type=jnp.float32)
    hid = jnp.maximum(hid + b1_ref[0][:, None], 0.0)
    ffn = jnp.dot(w2_ref[...], hid, preferred_element_type=jnp.float32)
    y_ref[0] = (ef + ffn + b2_ref[0][:, None]).reshape(_C, _R, _W)


def _col_resample_matrix():
    half = _W // 2
    d = np.zeros((half, _W), np.float32)
    for n in range(half):
        d[n, 2 * n] = 0.5
        d[n, 2 * n + 1] = 0.5
    u = np.zeros((_W, half), np.float32)
    for i in range(_W):
        src = (i + 0.5) * 0.5 - 0.5
        src = min(max(src, 0.0), half - 1.0)
        i0 = int(np.floor(src))
        i1 = min(i0 + 1, half - 1)
        w = src - i0
        u[i, i0] += 1.0 - w
        u[i, i1] += w
    return jnp.asarray((u @ d).T)


@jax.jit
def kernel(x, gn_gamma, gn_beta, w1, b1, w2, b2):
    mw = _col_resample_matrix()
    z2 = jnp.zeros((_B, _C, 2, _W), x.dtype)
    hal2 = jnp.stack(
        [jnp.concatenate(
            [x[:, :, t * _R - 2:t * _R] if t > 0 else z2,
             x[:, :, t * _R + _R:t * _R + _R + 2] if t < _T - 1 else z2],
            axis=2)
         for t in range(_T)], axis=1)

    df, stats = pl.pallas_call(
        _stats_body,
        grid=(_B, _T),
        in_specs=[
            pl.BlockSpec((1, _C, _R, _W), lambda b, t: (b, 0, t, 0)),
            pl.BlockSpec((1, 1, _C, 4, _W), lambda b, t: (b, t, 0, 0, 0)),
            pl.BlockSpec((_W, _W), lambda b, t: (0, 0)),
        ],
        out_specs=[
            pl.BlockSpec((1, _R, _W), lambda b, t: (b, t, 0)),
            pl.BlockSpec((1, 1, 1, _NSTAT), lambda b, t: (b, t, 0, 0)),
        ],
        out_shape=[
            jax.ShapeDtypeStruct((_B, _H, _W), jnp.float32),
            jax.ShapeDtypeStruct((_B, _T, 1, _NSTAT), jnp.float32),
        ],
    )(x, hal2, mw)
    stats = stats.reshape(_B, _T, _NSTAT)

    y = pl.pallas_call(
        _main_body,
        grid=(_B, _T),
        in_specs=[
            pl.BlockSpec((1, _C, _R, _W), lambda b, t: (b, 0, t, 0)),
            pl.BlockSpec((1, 1, _C, 4, _W), lambda b, t: (b, t, 0, 0, 0)),
            pl.BlockSpec((1, _R, _W), lambda b, t: (b, t, 0)),
            pl.BlockSpec((1, _T, _NSTAT), lambda b, t: (b, 0, 0)),
            pl.BlockSpec((_NG, _CPG), lambda b, t: (0, 0)),
            pl.BlockSpec((_NG, _CPG), lambda b, t: (0, 0)),
            pl.BlockSpec((2 * _C, _C), lambda b, t: (0, 0)),
            pl.BlockSpec((1, 2 * _C), lambda b, t: (0, 0)),
            pl.BlockSpec((_C, 2 * _C), lambda b, t: (0, 0)),
            pl.BlockSpec((1, _C), lambda b, t: (0, 0)),
        ],
        out_specs=pl.BlockSpec((1, _C, _R, _W), lambda b, t: (b, 0, t, 0)),
        out_shape=jax.ShapeDtypeStruct((_B, _C, _H, _W), jnp.float32),
    )(x, hal2, df, stats, gn_gamma.reshape(_NG, _CPG),
      gn_beta.reshape(_NG, _CPG), w1, b1.reshape(1, 2 * _C), w2,
      b2.reshape(1, _C))
    return y
